# baseline (speedup 1.0000x reference)
import functools

import jax
import jax.numpy as jnp
import numpy as np
from jax.experimental import pallas as pl
from jax.experimental.pallas import tpu as pltpu
from jax.experimental.pallas import tpu_sc as plsc

PLANES = [32, 64, 128, 256, 512]
STRIDE = [1, 4, 4, 4, 4]
NSAMPLE = [8, 16, 16, 16, 16]
SHARE = 8
N0 = 16384
IN_CH = 6
NUM_CLASSES = 50

_NW = 32
_CH = 128


def lin(x, p):
    return x @ p["w"] + p["b"]


def _knn_body(k, n, qb, q_ref, kpt_ref, o_ref):
    q = q_ref[...]
    kpt = kpt_ref[...]
    kn2 = jnp.sum(kpt * kpt, axis=0)[None, :]
    q2 = jnp.sum(q * q, axis=1, keepdims=True)
    d = q2 + kn2 - 2.0 * jnp.dot(q, kpt, preferred_element_type=jnp.float32)
    iota = jax.lax.broadcasted_iota(jnp.int32, (qb, n), 1)
    big = jnp.float32(jnp.inf)
    for j in range(k):
        m = jnp.min(d, axis=1, keepdims=True)
        cand = jnp.where(d == m, iota, n)
        ij = jnp.min(cand, axis=1, keepdims=True)
        o_ref[:, j] = ij[:, 0]
        d = jnp.where(iota == ij, big, d)


def knn_idx(q, kp, k):
    M, N = q.shape[0], kp.shape[0]
    QB = min(M, 256)
    qpad = jnp.pad(q, ((0, 0), (0, 5)))
    kpt = jnp.pad(kp, ((0, 0), (0, 5))).T
    body = functools.partial(_knn_body, k, N, QB)
    return pl.pallas_call(
        body,
        grid=(M // QB,),
        in_specs=[
            pl.BlockSpec((QB, 8), lambda i: (i, 0)),
            pl.BlockSpec((8, N), lambda i: (0, 0)),
        ],
        out_specs=pl.BlockSpec((QB, k), lambda i: (i, 0)),
        out_shape=jax.ShapeDtypeStruct((M, k), jnp.int32),
    )(qpad, kpt)


def _sc_gather_body(total_chunks, nct_max,
                    tab, idx2, out, idx_v, rows, sem):
    wid = jax.lax.axis_index("s") * 2 + jax.lax.axis_index("c")

    def step(j, carry):
        cid = wid + j * _NW

        @pl.when(cid < total_chunks)
        def _():
            pltpu.sync_copy(idx2.at[cid], idx_v)
            pltpu.async_copy(tab.at[idx_v], rows, sem).wait()
            pltpu.sync_copy(rows, out.at[pl.ds(cid * _CH, _CH)])

        return carry

    jax.lax.fori_loop(0, nct_max, step, 0)


def sc_gather(tab, idx):
    M = idx.size
    d = tab.shape[1]
    assert M % _CH == 0 and d % 128 == 0
    total_chunks = M // _CH
    nct_max = (total_chunks + _NW - 1) // _NW
    idx2 = idx.reshape(total_chunks, _CH)
    mesh = plsc.VectorSubcoreMesh(core_axis_name="c", subcore_axis_name="s")
    body = functools.partial(_sc_gather_body, total_chunks, nct_max)
    fn = pl.kernel(
        body,
        mesh=mesh,
        out_type=jax.ShapeDtypeStruct((M, d), jnp.float32),
        scratch_types=[
            pltpu.VMEM((_CH,), jnp.int32),
            pltpu.VMEM((_CH, d), jnp.float32),
            pltpu.SemaphoreType.DMA,
        ],
    )
    return fn(tab, idx2)


def _packed_table(feats, p):
    n, c = feats.shape
    dp = -(-(c + 3) // 128) * 128
    return jnp.pad(jnp.concatenate([feats, p], axis=1), ((0, 0), (0, dp - c - 3)))


def _pad16(p):
    return jnp.pad(p, ((0, 0), (0, 13)))


def _ptb_body(ns, c, s, b,
              x_ref, y_ref, g_ref, pb_ref,
              qw_ref, qb_ref, kw_ref, kb_ref, vw_ref, vb_ref,
              p1w_ref, p1b_ref, p2w_ref, p2b_ref,
              a1w_ref, a1b_ref, a2w_ref, a2b_ref,
              l2w_ref, l2b_ref, o_ref):
    f32 = jnp.float32
    q = jnp.dot(y_ref[...], qw_ref[...], preferred_element_type=f32) + qb_ref[...]
    gfull = g_ref[...]
    G = gfull[:, :, :c].reshape(b * ns, c)
    pr = (gfull[:, :, c:c + 16] - pb_ref[...][:, None, :]).reshape(b * ns, 16)
    kf = jnp.dot(G, kw_ref[...], preferred_element_type=f32) + kb_ref[...]
    v = jnp.dot(G, vw_ref[...], preferred_element_type=f32) + vb_ref[...]
    pe = jnp.maximum(
        jnp.dot(pr, p1w_ref[...], preferred_element_type=f32) + p1b_ref[...], 0.0)
    pe = jnp.dot(pe, p2w_ref[...], preferred_element_type=f32) + p2b_ref[...]
    w3 = q[:, None, :] - kf.reshape(b, ns, c) + pe.reshape(b, ns, c)
    h = jnp.maximum(
        jnp.dot(w3.reshape(b * ns, c), a1w_ref[...], preferred_element_type=f32)
        + a1b_ref[...], 0.0)
    h = jnp.dot(h, a2w_ref[...], preferred_element_type=f32) + a2b_ref[...]
    h3 = h.reshape(b, ns, s)
    m = jnp.max(h3, axis=1, keepdims=True)
    e = jnp.exp(h3 - m)
    z = jnp.sum(e, axis=1, keepdims=True)
    a = e / z
    af = jnp.concatenate([a] * SHARE, axis=2)
    vpe = v.reshape(b, ns, c) + pe.reshape(b, ns, c)
    attn = jnp.sum(vpe * af, axis=1)
    z2 = jnp.maximum(attn, 0.0)
    out = jnp.dot(z2, l2w_ref[...], preferred_element_type=f32) + l2b_ref[...]
    o_ref[...] = jnp.maximum(x_ref[...] + out, 0.0)


def pt_block(p, x, prm, ns, idx):
    n, c = x.shape
    s = c // SHARE
    lp = prm["layer"]
    y = jax.nn.relu(lin(x, prm["lin1"]))
    p16 = _pad16(p)
    tab = _packed_table(y, p)
    dp = tab.shape[1]
    g = sc_gather(tab, idx.reshape(-1)).reshape(n, ns, dp)
    p1w = jnp.pad(lp["p1"]["w"], ((0, 13), (0, 13)))
    p1b = jnp.pad(lp["p1"]["b"], ((0, 13),))
    p2w = jnp.pad(lp["p2"]["w"], ((0, 13), (0, 0)))
    b = min(n, 512)
    body = functools.partial(_ptb_body, ns, c, s, b)
    rep = lambda i: (0, 0)
    rep1 = lambda i: (0,)
    return pl.pallas_call(
        body,
        grid=(n // b,),
        in_specs=[
            pl.BlockSpec((b, c), lambda i: (i, 0)),
            pl.BlockSpec((b, c), lambda i: (i, 0)),
            pl.BlockSpec((b, ns, dp), lambda i: (i, 0, 0)),
            pl.BlockSpec((b, 16), lambda i: (i, 0)),
            pl.BlockSpec((c, c), rep), pl.BlockSpec((c,), rep1),
            pl.BlockSpec((c, c), rep), pl.BlockSpec((c,), rep1),
            pl.BlockSpec((c, c), rep), pl.BlockSpec((c,), rep1),
            pl.BlockSpec((16, 16), rep), pl.BlockSpec((16,), rep1),
            pl.BlockSpec((16, c), rep), pl.BlockSpec((c,), rep1),
            pl.BlockSpec((c, s), rep), pl.BlockSpec((s,), rep1),
            pl.BlockSpec((s, s), rep), pl.BlockSpec((s,), rep1),
            pl.BlockSpec((c, c), rep), pl.BlockSpec((c,), rep1),
        ],
        out_specs=pl.BlockSpec((b, c), lambda i: (i, 0)),
        out_shape=jax.ShapeDtypeStruct((n, c), jnp.float32),
    )(x, y, g, p16,
      lp["q"]["w"], lp["q"]["b"], lp["k"]["w"], lp["k"]["b"],
      lp["v"]["w"], lp["v"]["b"], p1w, p1b, p2w, lp["p2"]["b"],
      lp["a1"]["w"], lp["a1"]["b"], lp["a2"]["w"], lp["a2"]["b"],
      prm["lin2"]["w"], prm["lin2"]["b"])


def _tdown_body(ns, cin, cout, b,
                g_ref, pb_ref, wp_ref, wx_ref, b_ref, o_ref):
    f32 = jnp.float32
    gfull = g_ref[...]
    P = (gfull[:, :, cin:cin + 16] - pb_ref[...][:, None, :]).reshape(b * ns, 16)
    G = gfull[:, :, :cin].reshape(b * ns, cin)
    g = (jnp.dot(P, wp_ref[...], preferred_element_type=f32)
         + jnp.dot(G, wx_ref[...], preferred_element_type=f32) + b_ref[...])
    g = jnp.maximum(g, 0.0)
    o_ref[...] = jnp.max(g.reshape(b, ns, cout), axis=1)


def t_down(p, x, prm, stride, ns):
    if stride == 1:
        return p, jax.nn.relu(lin(x, prm))
    m = x.shape[0] // stride
    cin = x.shape[1]
    cout = prm["w"].shape[1]
    pn = p[:m]
    idx = knn_idx(pn, p, ns)
    p16 = _pad16(p)
    tab = _packed_table(x, p)
    dp = tab.shape[1]
    g = sc_gather(tab, idx.reshape(-1)).reshape(m, ns, dp)
    wp = jnp.pad(prm["w"][:3], ((0, 13), (0, 0)))
    wx = prm["w"][3:]
    b = min(m, 512)
    body = functools.partial(_tdown_body, ns, cin, cout, b)
    rep = lambda i: (0, 0)
    rep1 = lambda i: (0,)
    g = pl.pallas_call(
        body,
        grid=(m // b,),
        in_specs=[
            pl.BlockSpec((b, ns, dp), lambda i: (i, 0, 0)),
            pl.BlockSpec((b, 16), lambda i: (i, 0)),
            pl.BlockSpec((16, cout), rep),
            pl.BlockSpec((cin, cout), rep),
            pl.BlockSpec((cout,), rep1),
        ],
        out_specs=pl.BlockSpec((b, cout), lambda i: (i, 0)),
        out_shape=jax.ShapeDtypeStruct((m, cout), jnp.float32),
    )(g, p16[:m], wp, wx, prm["b"])
    return pn, g


def _tup_body(cf, b, xf_ref, g_ref, w_ref, l1w_ref, l1b_ref, o_ref):
    f32 = jnp.float32
    x1 = jnp.dot(xf_ref[...], l1w_ref[...], preferred_element_type=f32) + l1b_ref[...]
    xg = g_ref[...][:, :, :cf]
    w = w_ref[...]
    o_ref[...] = x1 + jnp.sum(xg * w[:, :, None], axis=1)


def t_up(pf, xf, pc, xc, prm):
    n, cf = xf.shape[0], prm["l1"]["w"].shape[1]
    x2 = lin(xc, prm["l2"])
    idx = knn_idx(pf, pc, 3)
    idx8 = jnp.pad(idx, ((0, 0), (0, 5)))
    tab = _packed_table(x2, pc)
    dp = tab.shape[1]
    g = sc_gather(tab, idx8.reshape(-1)).reshape(n, 8, dp)
    pcg = g[:, :3, cf:cf + 3]
    d = jnp.sum((pf[:, None, :] - pcg) ** 2, -1)
    w = 1.0 / (d + 1e-8)
    w = w / jnp.sum(w, -1, keepdims=True)
    w = jnp.pad(w, ((0, 0), (0, 5)))
    b = min(n, 512)
    body = functools.partial(_tup_body, cf, b)
    rep = lambda i: (0, 0)
    rep1 = lambda i: (0,)
    return pl.pallas_call(
        body,
        grid=(n // b,),
        in_specs=[
            pl.BlockSpec((b, cf), lambda i: (i, 0)),
            pl.BlockSpec((b, 8, dp), lambda i: (i, 0, 0)),
            pl.BlockSpec((b, 8), lambda i: (i, 0)),
            pl.BlockSpec((cf, cf), rep),
            pl.BlockSpec((cf,), rep1),
        ],
        out_specs=pl.BlockSpec((b, cf), lambda i: (i, 0)),
        out_shape=jax.ShapeDtypeStruct((n, cf), jnp.float32),
    )(xf, g, w, prm["l1"]["w"], prm["l1"]["b"])


def t_up_head(x, prm):
    x1 = lin(x, prm["l1"])
    g = lin(jnp.mean(x, axis=0, keepdims=True), prm["l2"])
    return x1 + g


def _cls_kernel(x_ref, w1_ref, b1_ref, w2_ref, b2_ref, o_ref):
    y = jnp.maximum(jnp.dot(x_ref[...], w1_ref[...],
                            preferred_element_type=jnp.float32) + b1_ref[...], 0.0)
    o_ref[...] = jnp.dot(y, w2_ref[...],
                         preferred_element_type=jnp.float32) + b2_ref[...]


def cls_head(x, p1, p2):
    n, c = x.shape
    nc = NUM_CLASSES
    blk = 2048
    return pl.pallas_call(
        _cls_kernel,
        grid=(n // blk,),
        in_specs=[
            pl.BlockSpec((blk, c), lambda i: (i, 0)),
            pl.BlockSpec((c, c), lambda i: (0, 0)),
            pl.BlockSpec((c,), lambda i: (0,)),
            pl.BlockSpec((c, nc), lambda i: (0, 0)),
            pl.BlockSpec((nc,), lambda i: (0,)),
        ],
        out_specs=pl.BlockSpec((blk, nc), lambda i: (i, 0)),
        out_shape=jax.ShapeDtypeStruct((n, nc), jnp.float32),
    )(x, p1["w"], p1["b"], p2["w"], p2["b"])


def kernel(coord, feat, offset, params):
    p1, x1 = t_down(coord, feat, params["enc1_td"], 1, NSAMPLE[0])
    idxs = [None] * 5
    idxs[0] = knn_idx(coord, coord, NSAMPLE[0])
    x1 = pt_block(p1, x1, params["enc1_blk"], NSAMPLE[0], idxs[0])
    ps, xs = [p1], [x1]
    pc, xc = p1, x1
    for i in range(1, 5):
        pc, xc = t_down(pc, xc, params["enc%d_td" % (i + 1)], STRIDE[i], NSAMPLE[i])
        idxs[i] = knn_idx(pc, pc, NSAMPLE[i])
        xc = pt_block(pc, xc, params["enc%d_blk" % (i + 1)], NSAMPLE[i], idxs[i])
        ps.append(pc)
        xs.append(xc)
    p1, p2, p3, p4, p5 = ps
    x1, x2, x3, x4, x5 = xs
    x5 = pt_block(p5, t_up_head(x5, params["dec5_tu"]), params["dec5_blk"], NSAMPLE[4], idxs[4])
    x4 = pt_block(p4, t_up(p4, x4, p5, x5, params["dec4_tu"]), params["dec4_blk"], NSAMPLE[3], idxs[3])
    x3 = pt_block(p3, t_up(p3, x3, p4, x4, params["dec3_tu"]), params["dec3_blk"], NSAMPLE[2], idxs[2])
    x2 = pt_block(p2, t_up(p2, x2, p3, x3, params["dec2_tu"]), params["dec2_blk"], NSAMPLE[1], idxs[1])
    x1 = pt_block(p1, t_up(p1, x1, p2, x2, params["dec1_tu"]), params["dec1_blk"], NSAMPLE[0], idxs[0])
    return cls_head(x1, params["cls1"], params["cls2"])


# speedup vs baseline: 1.0030x; 1.0030x over previous
import functools

import jax
import jax.numpy as jnp
import numpy as np
from jax.experimental import pallas as pl
from jax.experimental.pallas import tpu as pltpu
from jax.experimental.pallas import tpu_sc as plsc

PLANES = [32, 64, 128, 256, 512]
STRIDE = [1, 4, 4, 4, 4]
NSAMPLE = [8, 16, 16, 16, 16]
SHARE = 8
N0 = 16384
IN_CH = 6
NUM_CLASSES = 50

_NW = 32
_CH = 128


def lin(x, p):
    return x @ p["w"] + p["b"]


def _knn_body(k, n, qb, q_ref, kpt_ref, o_ref):
    q = q_ref[...]
    kpt = kpt_ref[...]
    kn2 = jnp.sum(kpt * kpt, axis=0)[None, :]
    q2 = jnp.sum(q * q, axis=1, keepdims=True)
    d = q2 + kn2 - 2.0 * jnp.dot(q, kpt, preferred_element_type=jnp.float32)
    iota = jax.lax.broadcasted_iota(jnp.int32, (qb, n), 1)
    big = jnp.float32(jnp.inf)
    for j in range(k):
        m = jnp.min(d, axis=1, keepdims=True)
        cand = jnp.where(d == m, iota, n)
        ij = jnp.min(cand, axis=1, keepdims=True)
        o_ref[:, j] = ij[:, 0]
        d = jnp.where(iota == ij, big, d)


def knn_idx(q, kp, k):
    M, N = q.shape[0], kp.shape[0]
    QB = min(M, 256)
    qpad = jnp.pad(q, ((0, 0), (0, 5)))
    kpt = jnp.pad(kp, ((0, 0), (0, 5))).T
    body = functools.partial(_knn_body, k, N, QB)
    return pl.pallas_call(
        body,
        grid=(M // QB,),
        in_specs=[
            pl.BlockSpec((QB, 8), lambda i: (i, 0)),
            pl.BlockSpec((8, N), lambda i: (0, 0)),
        ],
        out_specs=pl.BlockSpec((QB, k), lambda i: (i, 0)),
        out_shape=jax.ShapeDtypeStruct((M, k), jnp.int32),
    )(qpad, kpt)


def _sc_gather_body(tc, nct, ring, tab, idx2, out, idx_v, bufs, sg, ss):
    wid = jax.lax.axis_index("s") * 2 + jax.lax.axis_index("c")
    base = wid * nct

    @pl.when(base < tc)
    def _():
        pltpu.sync_copy(idx2.at[pl.ds(base, nct)], idx_v)

    ngroups = (nct + ring - 1) // ring

    def group(g, carry):
        j0 = g * ring
        copies = []
        for b in range(ring):
            cid = base + j0 + b

            @pl.when(jnp.logical_and(j0 + b < nct, cid < tc))
            def _(b=b, cid=cid):
                pltpu.async_copy(tab.at[idx_v.at[j0 + b]], bufs[b], sg)

        for b in range(ring):
            cid = base + j0 + b

            @pl.when(jnp.logical_and(j0 + b < nct, cid < tc))
            def _(b=b, cid=cid):
                pltpu.make_async_copy(tab.at[idx_v.at[j0 + b]], bufs[b], sg).wait()
                pltpu.async_copy(bufs[b], out.at[pl.ds(cid * _CH, _CH)], ss)

        for b in range(ring):
            cid = base + j0 + b

            @pl.when(jnp.logical_and(j0 + b < nct, cid < tc))
            def _(b=b, cid=cid):
                pltpu.make_async_copy(bufs[b], out.at[pl.ds(cid * _CH, _CH)], ss).wait()

        return carry

    @pl.when(base < tc)
    def _():
        jax.lax.fori_loop(0, ngroups, group, 0)


def sc_gather(tab, idx):
    M = idx.size
    d = tab.shape[1]
    assert M % _CH == 0 and d % 128 == 0
    tc = M // _CH
    nct = (tc + _NW - 1) // _NW
    ring = max(1, min(4, 262144 // (_CH * d * 4), nct))
    idx2 = jnp.pad(idx.reshape(tc, _CH), ((0, _NW * nct - tc), (0, 0)))
    mesh = plsc.VectorSubcoreMesh(core_axis_name="c", subcore_axis_name="s")
    body = functools.partial(_sc_gather_body, tc, nct, ring)
    fn = pl.kernel(
        body,
        mesh=mesh,
        out_type=jax.ShapeDtypeStruct((M, d), jnp.float32),
        scratch_types=[
            pltpu.VMEM((nct, _CH), jnp.int32),
            [pltpu.VMEM((_CH, d), jnp.float32) for _ in range(ring)],
            pltpu.SemaphoreType.DMA,
            pltpu.SemaphoreType.DMA,
        ],
    )
    return fn(tab, idx2)


def _packed_table(feats, p):
    n, c = feats.shape
    dp = -(-(c + 3) // 128) * 128
    return jnp.pad(jnp.concatenate([feats, p], axis=1), ((0, 0), (0, dp - c - 3)))


def _pad16(p):
    return jnp.pad(p, ((0, 0), (0, 13)))


def _ptb_body(ns, c, s, b,
              x_ref, y_ref, g_ref, pb_ref,
              qw_ref, qb_ref, kw_ref, kb_ref, vw_ref, vb_ref,
              p1w_ref, p1b_ref, p2w_ref, p2b_ref,
              a1w_ref, a1b_ref, a2w_ref, a2b_ref,
              l2w_ref, l2b_ref, o_ref):
    f32 = jnp.float32
    q = jnp.dot(y_ref[...], qw_ref[...], preferred_element_type=f32) + qb_ref[...]
    gfull = g_ref[...]
    G = gfull[:, :, :c].reshape(b * ns, c)
    pr = (gfull[:, :, c:c + 16] - pb_ref[...][:, None, :]).reshape(b * ns, 16)
    kf = jnp.dot(G, kw_ref[...], preferred_element_type=f32) + kb_ref[...]
    v = jnp.dot(G, vw_ref[...], preferred_element_type=f32) + vb_ref[...]
    pe = jnp.maximum(
        jnp.dot(pr, p1w_ref[...], preferred_element_type=f32) + p1b_ref[...], 0.0)
    pe = jnp.dot(pe, p2w_ref[...], preferred_element_type=f32) + p2b_ref[...]
    w3 = q[:, None, :] - kf.reshape(b, ns, c) + pe.reshape(b, ns, c)
    h = jnp.maximum(
        jnp.dot(w3.reshape(b * ns, c), a1w_ref[...], preferred_element_type=f32)
        + a1b_ref[...], 0.0)
    h = jnp.dot(h, a2w_ref[...], preferred_element_type=f32) + a2b_ref[...]
    h3 = h.reshape(b, ns, s)
    m = jnp.max(h3, axis=1, keepdims=True)
    e = jnp.exp(h3 - m)
    z = jnp.sum(e, axis=1, keepdims=True)
    a = e / z
    af = jnp.concatenate([a] * SHARE, axis=2)
    vpe = v.reshape(b, ns, c) + pe.reshape(b, ns, c)
    attn = jnp.sum(vpe * af, axis=1)
    z2 = jnp.maximum(attn, 0.0)
    out = jnp.dot(z2, l2w_ref[...], preferred_element_type=f32) + l2b_ref[...]
    o_ref[...] = jnp.maximum(x_ref[...] + out, 0.0)


def pt_block(p, x, prm, ns, idx):
    n, c = x.shape
    s = c // SHARE
    lp = prm["layer"]
    y = jax.nn.relu(lin(x, prm["lin1"]))
    p16 = _pad16(p)
    tab = _packed_table(y, p)
    dp = tab.shape[1]
    g = sc_gather(tab, idx.reshape(-1)).reshape(n, ns, dp)
    p1w = jnp.pad(lp["p1"]["w"], ((0, 13), (0, 13)))
    p1b = jnp.pad(lp["p1"]["b"], ((0, 13),))
    p2w = jnp.pad(lp["p2"]["w"], ((0, 13), (0, 0)))
    b = min(n, 512)
    body = functools.partial(_ptb_body, ns, c, s, b)
    rep = lambda i: (0, 0)
    rep1 = lambda i: (0,)
    return pl.pallas_call(
        body,
        grid=(n // b,),
        in_specs=[
            pl.BlockSpec((b, c), lambda i: (i, 0)),
            pl.BlockSpec((b, c), lambda i: (i, 0)),
            pl.BlockSpec((b, ns, dp), lambda i: (i, 0, 0)),
            pl.BlockSpec((b, 16), lambda i: (i, 0)),
            pl.BlockSpec((c, c), rep), pl.BlockSpec((c,), rep1),
            pl.BlockSpec((c, c), rep), pl.BlockSpec((c,), rep1),
            pl.BlockSpec((c, c), rep), pl.BlockSpec((c,), rep1),
            pl.BlockSpec((16, 16), rep), pl.BlockSpec((16,), rep1),
            pl.BlockSpec((16, c), rep), pl.BlockSpec((c,), rep1),
            pl.BlockSpec((c, s), rep), pl.BlockSpec((s,), rep1),
            pl.BlockSpec((s, s), rep), pl.BlockSpec((s,), rep1),
            pl.BlockSpec((c, c), rep), pl.BlockSpec((c,), rep1),
        ],
        out_specs=pl.BlockSpec((b, c), lambda i: (i, 0)),
        out_shape=jax.ShapeDtypeStruct((n, c), jnp.float32),
    )(x, y, g, p16,
      lp["q"]["w"], lp["q"]["b"], lp["k"]["w"], lp["k"]["b"],
      lp["v"]["w"], lp["v"]["b"], p1w, p1b, p2w, lp["p2"]["b"],
      lp["a1"]["w"], lp["a1"]["b"], lp["a2"]["w"], lp["a2"]["b"],
      prm["lin2"]["w"], prm["lin2"]["b"])


def _tdown_body(ns, cin, cout, b,
                g_ref, pb_ref, wp_ref, wx_ref, b_ref, o_ref):
    f32 = jnp.float32
    gfull = g_ref[...]
    P = (gfull[:, :, cin:cin + 16] - pb_ref[...][:, None, :]).reshape(b * ns, 16)
    G = gfull[:, :, :cin].reshape(b * ns, cin)
    g = (jnp.dot(P, wp_ref[...], preferred_element_type=f32)
         + jnp.dot(G, wx_ref[...], preferred_element_type=f32) + b_ref[...])
    g = jnp.maximum(g, 0.0)
    o_ref[...] = jnp.max(g.reshape(b, ns, cout), axis=1)


def t_down(p, x, prm, stride, ns):
    if stride == 1:
        return p, jax.nn.relu(lin(x, prm))
    m = x.shape[0] // stride
    cin = x.shape[1]
    cout = prm["w"].shape[1]
    pn = p[:m]
    idx = knn_idx(pn, p, ns)
    p16 = _pad16(p)
    tab = _packed_table(x, p)
    dp = tab.shape[1]
    g = sc_gather(tab, idx.reshape(-1)).reshape(m, ns, dp)
    wp = jnp.pad(prm["w"][:3], ((0, 13), (0, 0)))
    wx = prm["w"][3:]
    b = min(m, 512)
    body = functools.partial(_tdown_body, ns, cin, cout, b)
    rep = lambda i: (0, 0)
    rep1 = lambda i: (0,)
    g = pl.pallas_call(
        body,
        grid=(m // b,),
        in_specs=[
            pl.BlockSpec((b, ns, dp), lambda i: (i, 0, 0)),
            pl.BlockSpec((b, 16), lambda i: (i, 0)),
            pl.BlockSpec((16, cout), rep),
            pl.BlockSpec((cin, cout), rep),
            pl.BlockSpec((cout,), rep1),
        ],
        out_specs=pl.BlockSpec((b, cout), lambda i: (i, 0)),
        out_shape=jax.ShapeDtypeStruct((m, cout), jnp.float32),
    )(g, p16[:m], wp, wx, prm["b"])
    return pn, g


def _tup_body(cf, b, xf_ref, g_ref, w_ref, l1w_ref, l1b_ref, o_ref):
    f32 = jnp.float32
    x1 = jnp.dot(xf_ref[...], l1w_ref[...], preferred_element_type=f32) + l1b_ref[...]
    xg = g_ref[...][:, :, :cf]
    w = w_ref[...]
    o_ref[...] = x1 + jnp.sum(xg * w[:, :, None], axis=1)


def t_up(pf, xf, pc, xc, prm):
    n, cf = xf.shape[0], prm["l1"]["w"].shape[1]
    x2 = lin(xc, prm["l2"])
    idx = knn_idx(pf, pc, 3)
    idx8 = jnp.pad(idx, ((0, 0), (0, 5)))
    tab = _packed_table(x2, pc)
    dp = tab.shape[1]
    g = sc_gather(tab, idx8.reshape(-1)).reshape(n, 8, dp)
    pcg = g[:, :3, cf:cf + 3]
    d = jnp.sum((pf[:, None, :] - pcg) ** 2, -1)
    w = 1.0 / (d + 1e-8)
    w = w / jnp.sum(w, -1, keepdims=True)
    w = jnp.pad(w, ((0, 0), (0, 5)))
    b = min(n, 512)
    body = functools.partial(_tup_body, cf, b)
    rep = lambda i: (0, 0)
    rep1 = lambda i: (0,)
    return pl.pallas_call(
        body,
        grid=(n // b,),
        in_specs=[
            pl.BlockSpec((b, cf), lambda i: (i, 0)),
            pl.BlockSpec((b, 8, dp), lambda i: (i, 0, 0)),
            pl.BlockSpec((b, 8), lambda i: (i, 0)),
            pl.BlockSpec((cf, cf), rep),
            pl.BlockSpec((cf,), rep1),
        ],
        out_specs=pl.BlockSpec((b, cf), lambda i: (i, 0)),
        out_shape=jax.ShapeDtypeStruct((n, cf), jnp.float32),
    )(xf, g, w, prm["l1"]["w"], prm["l1"]["b"])


def t_up_head(x, prm):
    x1 = lin(x, prm["l1"])
    g = lin(jnp.mean(x, axis=0, keepdims=True), prm["l2"])
    return x1 + g


def _cls_kernel(x_ref, w1_ref, b1_ref, w2_ref, b2_ref, o_ref):
    y = jnp.maximum(jnp.dot(x_ref[...], w1_ref[...],
                            preferred_element_type=jnp.float32) + b1_ref[...], 0.0)
    o_ref[...] = jnp.dot(y, w2_ref[...],
                         preferred_element_type=jnp.float32) + b2_ref[...]


def cls_head(x, p1, p2):
    n, c = x.shape
    nc = NUM_CLASSES
    blk = 2048
    return pl.pallas_call(
        _cls_kernel,
        grid=(n // blk,),
        in_specs=[
            pl.BlockSpec((blk, c), lambda i: (i, 0)),
            pl.BlockSpec((c, c), lambda i: (0, 0)),
            pl.BlockSpec((c,), lambda i: (0,)),
            pl.BlockSpec((c, nc), lambda i: (0, 0)),
            pl.BlockSpec((nc,), lambda i: (0,)),
        ],
        out_specs=pl.BlockSpec((blk, nc), lambda i: (i, 0)),
        out_shape=jax.ShapeDtypeStruct((n, nc), jnp.float32),
    )(x, p1["w"], p1["b"], p2["w"], p2["b"])


def kernel(coord, feat, offset, params):
    p1, x1 = t_down(coord, feat, params["enc1_td"], 1, NSAMPLE[0])
    idxs = [None] * 5
    idxs[0] = knn_idx(coord, coord, NSAMPLE[0])
    x1 = pt_block(p1, x1, params["enc1_blk"], NSAMPLE[0], idxs[0])
    ps, xs = [p1], [x1]
    pc, xc = p1, x1
    for i in range(1, 5):
        pc, xc = t_down(pc, xc, params["enc%d_td" % (i + 1)], STRIDE[i], NSAMPLE[i])
        idxs[i] = knn_idx(pc, pc, NSAMPLE[i])
        xc = pt_block(pc, xc, params["enc%d_blk" % (i + 1)], NSAMPLE[i], idxs[i])
        ps.append(pc)
        xs.append(xc)
    p1, p2, p3, p4, p5 = ps
    x1, x2, x3, x4, x5 = xs
    x5 = pt_block(p5, t_up_head(x5, params["dec5_tu"]), params["dec5_blk"], NSAMPLE[4], idxs[4])
    x4 = pt_block(p4, t_up(p4, x4, p5, x5, params["dec4_tu"]), params["dec4_blk"], NSAMPLE[3], idxs[3])
    x3 = pt_block(p3, t_up(p3, x3, p4, x4, params["dec3_tu"]), params["dec3_blk"], NSAMPLE[2], idxs[2])
    x2 = pt_block(p2, t_up(p2, x2, p3, x3, params["dec2_tu"]), params["dec2_blk"], NSAMPLE[1], idxs[1])
    x1 = pt_block(p1, t_up(p1, x1, p2, x2, params["dec1_tu"]), params["dec1_blk"], NSAMPLE[0], idxs[0])
    return cls_head(x1, params["cls1"], params["cls2"])


# speedup vs baseline: 1.1259x; 1.1225x over previous
import functools

import jax
import jax.numpy as jnp
import numpy as np
from jax.experimental import pallas as pl

PLANES = [32, 64, 128, 256, 512]
STRIDE = [1, 4, 4, 4, 4]
NSAMPLE = [8, 16, 16, 16, 16]
SHARE = 8
N0 = 16384
IN_CH = 6
NUM_CLASSES = 50


def lin(x, p):
    return x @ p["w"] + p["b"]


def _knn_body(k, n, qb, q_ref, kpt_ref, o_ref):
    q = q_ref[...]
    kpt = kpt_ref[...]
    kn2 = jnp.sum(kpt * kpt, axis=0)[None, :]
    q2 = jnp.sum(q * q, axis=1, keepdims=True)
    d = q2 + kn2 - 2.0 * jnp.dot(q, kpt, preferred_element_type=jnp.float32)
    iota = jax.lax.broadcasted_iota(jnp.int32, (qb, n), 1)
    big = jnp.float32(jnp.inf)
    for j in range(k):
        m = jnp.min(d, axis=1, keepdims=True)
        cand = jnp.where(d == m, iota, n)
        ij = jnp.min(cand, axis=1, keepdims=True)
        o_ref[:, j] = ij[:, 0]
        d = jnp.where(iota == ij, big, d)


def knn_idx(q, kp, k):
    M, N = q.shape[0], kp.shape[0]
    QB = min(M, 256)
    qpad = jnp.pad(q, ((0, 0), (0, 5)))
    kpt = jnp.pad(kp, ((0, 0), (0, 5))).T
    body = functools.partial(_knn_body, k, N, QB)
    return pl.pallas_call(
        body,
        grid=(M // QB,),
        in_specs=[
            pl.BlockSpec((QB, 8), lambda i: (i, 0)),
            pl.BlockSpec((8, N), lambda i: (0, 0)),
        ],
        out_specs=pl.BlockSpec((QB, k), lambda i: (i, 0)),
        out_shape=jax.ShapeDtypeStruct((M, k), jnp.int32),
    )(qpad, kpt)


def _ptb_body(ns, c, s, b,
              x_ref, y_ref, yg_ref, pr_ref,
              qw_ref, qb_ref, kw_ref, kb_ref, vw_ref, vb_ref,
              p1w_ref, p1b_ref, p2w_ref, p2b_ref,
              a1w_ref, a1b_ref, a2w_ref, a2b_ref,
              l2w_ref, l2b_ref, o_ref):
    f32 = jnp.float32
    q = jnp.dot(y_ref[...], qw_ref[...], preferred_element_type=f32) + qb_ref[...]
    G = yg_ref[...].reshape(b * ns, c)
    P = pr_ref[...].reshape(b * ns, 8)
    kf = jnp.dot(G, kw_ref[...], preferred_element_type=f32) + kb_ref[...]
    v = jnp.dot(G, vw_ref[...], preferred_element_type=f32) + vb_ref[...]
    pe = jnp.maximum(
        jnp.dot(P, p1w_ref[...], preferred_element_type=f32) + p1b_ref[...], 0.0)
    pe = jnp.dot(pe, p2w_ref[...], preferred_element_type=f32) + p2b_ref[...]
    w3 = q[:, None, :] - kf.reshape(b, ns, c) + pe.reshape(b, ns, c)
    h = jnp.maximum(
        jnp.dot(w3.reshape(b * ns, c), a1w_ref[...], preferred_element_type=f32)
        + a1b_ref[...], 0.0)
    h = jnp.dot(h, a2w_ref[...], preferred_element_type=f32) + a2b_ref[...]
    h3 = h.reshape(b, ns, s)
    m = jnp.max(h3, axis=1, keepdims=True)
    e = jnp.exp(h3 - m)
    z = jnp.sum(e, axis=1, keepdims=True)
    a = e / z
    af = jnp.concatenate([a] * SHARE, axis=2)
    vpe = v.reshape(b, ns, c) + pe.reshape(b, ns, c)
    attn = jnp.sum(vpe * af, axis=1)
    z2 = jnp.maximum(attn, 0.0)
    out = jnp.dot(z2, l2w_ref[...], preferred_element_type=f32) + l2b_ref[...]
    o_ref[...] = jnp.maximum(x_ref[...] + out, 0.0)


def pt_block(p, x, prm, ns, idx):
    n, c = x.shape
    s = c // SHARE
    lp = prm["layer"]
    y = jax.nn.relu(lin(x, prm["lin1"]))
    yg = y[idx]
    pr = p[idx] - p[:, None, :]
    pr = jnp.pad(pr, ((0, 0), (0, 0), (0, 5)))
    p1w = jnp.pad(lp["p1"]["w"], ((0, 5), (0, 5)))
    p1b = jnp.pad(lp["p1"]["b"], ((0, 5),))
    p2w = jnp.pad(lp["p2"]["w"], ((0, 5), (0, 0)))
    b = min(n, 512)
    body = functools.partial(_ptb_body, ns, c, s, b)
    rep = lambda i: (0, 0)
    rep1 = lambda i: (0,)
    return pl.pallas_call(
        body,
        grid=(n // b,),
        in_specs=[
            pl.BlockSpec((b, c), lambda i: (i, 0)),
            pl.BlockSpec((b, c), lambda i: (i, 0)),
            pl.BlockSpec((b, ns, c), lambda i: (i, 0, 0)),
            pl.BlockSpec((b, ns, 8), lambda i: (i, 0, 0)),
            pl.BlockSpec((c, c), rep), pl.BlockSpec((c,), rep1),
            pl.BlockSpec((c, c), rep), pl.BlockSpec((c,), rep1),
            pl.BlockSpec((c, c), rep), pl.BlockSpec((c,), rep1),
            pl.BlockSpec((8, 8), rep), pl.BlockSpec((8,), rep1),
            pl.BlockSpec((8, c), rep), pl.BlockSpec((c,), rep1),
            pl.BlockSpec((c, s), rep), pl.BlockSpec((s,), rep1),
            pl.BlockSpec((s, s), rep), pl.BlockSpec((s,), rep1),
            pl.BlockSpec((c, c), rep), pl.BlockSpec((c,), rep1),
        ],
        out_specs=pl.BlockSpec((b, c), lambda i: (i, 0)),
        out_shape=jax.ShapeDtypeStruct((n, c), jnp.float32),
    )(x, y, yg, pr,
      lp["q"]["w"], lp["q"]["b"], lp["k"]["w"], lp["k"]["b"],
      lp["v"]["w"], lp["v"]["b"], p1w, p1b, p2w, lp["p2"]["b"],
      lp["a1"]["w"], lp["a1"]["b"], lp["a2"]["w"], lp["a2"]["b"],
      prm["lin2"]["w"], prm["lin2"]["b"])


def _tdown_body(ns, cin, cout, b,
                pg_ref, xg_ref, wp_ref, wx_ref, b_ref, o_ref):
    f32 = jnp.float32
    P = pg_ref[...].reshape(b * ns, 8)
    G = xg_ref[...].reshape(b * ns, cin)
    g = (jnp.dot(P, wp_ref[...], preferred_element_type=f32)
         + jnp.dot(G, wx_ref[...], preferred_element_type=f32) + b_ref[...])
    g = jnp.maximum(g, 0.0)
    o_ref[...] = jnp.max(g.reshape(b, ns, cout), axis=1)


def t_down(p, x, prm, stride, ns):
    if stride == 1:
        return p, jax.nn.relu(lin(x, prm))
    m = x.shape[0] // stride
    cin = x.shape[1]
    cout = prm["w"].shape[1]
    pn = p[:m]
    idx = knn_idx(pn, p, ns)
    pg = p[idx] - pn[:, None, :]
    pg = jnp.pad(pg, ((0, 0), (0, 0), (0, 5)))
    xg = x[idx]
    wp = jnp.pad(prm["w"][:3], ((0, 5), (0, 0)))
    wx = prm["w"][3:]
    b = min(m, 512)
    body = functools.partial(_tdown_body, ns, cin, cout, b)
    rep = lambda i: (0, 0)
    rep1 = lambda i: (0,)
    g = pl.pallas_call(
        body,
        grid=(m // b,),
        in_specs=[
            pl.BlockSpec((b, ns, 8), lambda i: (i, 0, 0)),
            pl.BlockSpec((b, ns, cin), lambda i: (i, 0, 0)),
            pl.BlockSpec((8, cout), rep),
            pl.BlockSpec((cin, cout), rep),
            pl.BlockSpec((cout,), rep1),
        ],
        out_specs=pl.BlockSpec((b, cout), lambda i: (i, 0)),
        out_shape=jax.ShapeDtypeStruct((m, cout), jnp.float32),
    )(pg, xg, wp, wx, prm["b"])
    return pn, g


def _tup_body(cf, b, xf_ref, xg_ref, w_ref, l1w_ref, l1b_ref, o_ref):
    f32 = jnp.float32
    x1 = jnp.dot(xf_ref[...], l1w_ref[...], preferred_element_type=f32) + l1b_ref[...]
    xg = xg_ref[...]
    w = w_ref[...]
    o_ref[...] = x1 + jnp.sum(xg * w[:, :, None], axis=1)


def t_up(pf, xf, pc, xc, prm):
    n, cf = xf.shape[0], prm["l1"]["w"].shape[1]
    x2 = lin(xc, prm["l2"])
    idx = knn_idx(pf, pc, 3)
    d = jnp.sum((pf[:, None, :] - pc[idx]) ** 2, -1)
    w = 1.0 / (d + 1e-8)
    w = w / jnp.sum(w, -1, keepdims=True)
    w = jnp.pad(w, ((0, 0), (0, 5)))
    xg = jnp.pad(x2[idx], ((0, 0), (0, 5), (0, 0)))
    b = min(n, 512)
    body = functools.partial(_tup_body, cf, b)
    rep = lambda i: (0, 0)
    rep1 = lambda i: (0,)
    return pl.pallas_call(
        body,
        grid=(n // b,),
        in_specs=[
            pl.BlockSpec((b, cf), lambda i: (i, 0)),
            pl.BlockSpec((b, 8, cf), lambda i: (i, 0, 0)),
            pl.BlockSpec((b, 8), lambda i: (i, 0)),
            pl.BlockSpec((cf, cf), rep),
            pl.BlockSpec((cf,), rep1),
        ],
        out_specs=pl.BlockSpec((b, cf), lambda i: (i, 0)),
        out_shape=jax.ShapeDtypeStruct((n, cf), jnp.float32),
    )(xf, xg, w, prm["l1"]["w"], prm["l1"]["b"])


def t_up_head(x, prm):
    x1 = lin(x, prm["l1"])
    g = lin(jnp.mean(x, axis=0, keepdims=True), prm["l2"])
    return x1 + g


def _cls_kernel(x_ref, w1_ref, b1_ref, w2_ref, b2_ref, o_ref):
    y = jnp.maximum(jnp.dot(x_ref[...], w1_ref[...],
                            preferred_element_type=jnp.float32) + b1_ref[...], 0.0)
    o_ref[...] = jnp.dot(y, w2_ref[...],
                         preferred_element_type=jnp.float32) + b2_ref[...]


def cls_head(x, p1, p2):
    n, c = x.shape
    nc = NUM_CLASSES
    blk = 2048
    return pl.pallas_call(
        _cls_kernel,
        grid=(n // blk,),
        in_specs=[
            pl.BlockSpec((blk, c), lambda i: (i, 0)),
            pl.BlockSpec((c, c), lambda i: (0, 0)),
            pl.BlockSpec((c,), lambda i: (0,)),
            pl.BlockSpec((c, nc), lambda i: (0, 0)),
            pl.BlockSpec((nc,), lambda i: (0,)),
        ],
        out_specs=pl.BlockSpec((blk, nc), lambda i: (i, 0)),
        out_shape=jax.ShapeDtypeStruct((n, nc), jnp.float32),
    )(x, p1["w"], p1["b"], p2["w"], p2["b"])


def kernel(coord, feat, offset, params):
    p1, x1 = t_down(coord, feat, params["enc1_td"], 1, NSAMPLE[0])
    idxs = [knn_idx(coord, coord, NSAMPLE[0])] + [None] * 4
    x1 = pt_block(p1, x1, params["enc1_blk"], NSAMPLE[0], idxs[0])
    ps, xs = [p1], [x1]
    pc, xc = p1, x1
    for i in range(1, 5):
        pc, xc = t_down(pc, xc, params["enc%d_td" % (i + 1)], STRIDE[i], NSAMPLE[i])
        idxs[i] = knn_idx(pc, pc, NSAMPLE[i])
        xc = pt_block(pc, xc, params["enc%d_blk" % (i + 1)], NSAMPLE[i], idxs[i])
        ps.append(pc)
        xs.append(xc)
    p1, p2, p3, p4, p5 = ps
    x1, x2, x3, x4, x5 = xs
    x5 = pt_block(p5, t_up_head(x5, params["dec5_tu"]), params["dec5_blk"], NSAMPLE[4], idxs[4])
    x4 = pt_block(p4, t_up(p4, x4, p5, x5, params["dec4_tu"]), params["dec4_blk"], NSAMPLE[3], idxs[3])
    x3 = pt_block(p3, t_up(p3, x3, p4, x4, params["dec3_tu"]), params["dec3_blk"], NSAMPLE[2], idxs[2])
    x2 = pt_block(p2, t_up(p2, x2, p3, x3, params["dec2_tu"]), params["dec2_blk"], NSAMPLE[1], idxs[1])
    x1 = pt_block(p1, t_up(p1, x1, p2, x2, params["dec1_tu"]), params["dec1_blk"], NSAMPLE[0], idxs[0])
    return cls_head(x1, params["cls1"], params["cls2"])


# speedup vs baseline: 1.2963x; 1.1513x over previous
import functools

import jax
import jax.numpy as jnp
import numpy as np
from jax.experimental import pallas as pl

PLANES = [32, 64, 128, 256, 512]
STRIDE = [1, 4, 4, 4, 4]
NSAMPLE = [8, 16, 16, 16, 16]
SHARE = 8
N0 = 16384
IN_CH = 6
NUM_CLASSES = 50


def lin(x, p):
    return x @ p["w"] + p["b"]


def _knn_body(k, n, qb, q_ref, kpt_ref, o_ref):
    q = q_ref[...]
    kpt = kpt_ref[...]
    kn2 = jnp.sum(kpt * kpt, axis=0)[None, :]
    q2 = jnp.sum(q * q, axis=1, keepdims=True)
    d = q2 + kn2 - 2.0 * jnp.dot(q, kpt, preferred_element_type=jnp.float32)
    iota = jax.lax.broadcasted_iota(jnp.int32, (qb, n), 1)
    big = jnp.float32(jnp.inf)
    for j in range(k):
        m = jnp.min(d, axis=1, keepdims=True)
        cand = jnp.where(d == m, iota, n)
        ij = jnp.min(cand, axis=1, keepdims=True)
        o_ref[:, j] = ij[:, 0]
        d = jnp.where(iota == ij, big, d)


def knn_idx(q, kp, k):
    M, N = q.shape[0], kp.shape[0]
    QB = min(M, 256)
    qpad = jnp.pad(q, ((0, 0), (0, 5)))
    kpt = jnp.pad(kp, ((0, 0), (0, 5))).T
    body = functools.partial(_knn_body, k, N, QB)
    return pl.pallas_call(
        body,
        grid=(M // QB,),
        in_specs=[
            pl.BlockSpec((QB, 8), lambda i: (i, 0)),
            pl.BlockSpec((8, N), lambda i: (0, 0)),
        ],
        out_specs=pl.BlockSpec((QB, k), lambda i: (i, 0)),
        out_shape=jax.ShapeDtypeStruct((M, k), jnp.int32),
    )(qpad, kpt)


def _ptb_body(ns, c, s, b,
              x_ref, y_ref, yg_ref, pr_ref,
              qw_ref, qb_ref, kw_ref, kb_ref, vw_ref, vb_ref,
              p1w_ref, p1b_ref, p2w_ref, p2b_ref,
              a1w_ref, a1b_ref, a2w_ref, a2b_ref,
              l2w_ref, l2b_ref, o_ref):
    f32 = jnp.float32
    q = jnp.dot(y_ref[...], qw_ref[...], preferred_element_type=f32) + qb_ref[...]
    gfull = yg_ref[...]
    G = gfull[:, :, :c].reshape(b * ns, c)
    P = (gfull[:, :, c:c + 8] - pr_ref[...][:, None, :]).reshape(b * ns, 8)
    kf = jnp.dot(G, kw_ref[...], preferred_element_type=f32) + kb_ref[...]
    v = jnp.dot(G, vw_ref[...], preferred_element_type=f32) + vb_ref[...]
    pe = jnp.maximum(
        jnp.dot(P, p1w_ref[...], preferred_element_type=f32) + p1b_ref[...], 0.0)
    pe = jnp.dot(pe, p2w_ref[...], preferred_element_type=f32) + p2b_ref[...]
    w3 = q[:, None, :] - kf.reshape(b, ns, c) + pe.reshape(b, ns, c)
    h = jnp.maximum(
        jnp.dot(w3.reshape(b * ns, c), a1w_ref[...], preferred_element_type=f32)
        + a1b_ref[...], 0.0)
    h = jnp.dot(h, a2w_ref[...], preferred_element_type=f32) + a2b_ref[...]
    h3 = h.reshape(b, ns, s)
    m = jnp.max(h3, axis=1, keepdims=True)
    e = jnp.exp(h3 - m)
    z = jnp.sum(e, axis=1, keepdims=True)
    a = e / z
    af = jnp.concatenate([a] * SHARE, axis=2)
    vpe = v.reshape(b, ns, c) + pe.reshape(b, ns, c)
    attn = jnp.sum(vpe * af, axis=1)
    z2 = jnp.maximum(attn, 0.0)
    out = jnp.dot(z2, l2w_ref[...], preferred_element_type=f32) + l2b_ref[...]
    o_ref[...] = jnp.maximum(x_ref[...] + out, 0.0)


def pt_block(p, x, prm, ns, idx):
    n, c = x.shape
    s = c // SHARE
    lp = prm["layer"]
    y = jax.nn.relu(lin(x, prm["lin1"]))
    tab = jnp.pad(jnp.concatenate([y, p], axis=1), ((0, 0), (0, 5)))
    g = tab[idx]
    p8 = jnp.pad(p, ((0, 0), (0, 5)))
    p1w = jnp.pad(lp["p1"]["w"], ((0, 5), (0, 5)))
    p1b = jnp.pad(lp["p1"]["b"], ((0, 5),))
    p2w = jnp.pad(lp["p2"]["w"], ((0, 5), (0, 0)))
    b = min(n, 512)
    body = functools.partial(_ptb_body, ns, c, s, b)
    rep = lambda i: (0, 0)
    rep1 = lambda i: (0,)
    return pl.pallas_call(
        body,
        grid=(n // b,),
        in_specs=[
            pl.BlockSpec((b, c), lambda i: (i, 0)),
            pl.BlockSpec((b, c), lambda i: (i, 0)),
            pl.BlockSpec((b, ns, c + 8), lambda i: (i, 0, 0)),
            pl.BlockSpec((b, 8), lambda i: (i, 0)),
            pl.BlockSpec((c, c), rep), pl.BlockSpec((c,), rep1),
            pl.BlockSpec((c, c), rep), pl.BlockSpec((c,), rep1),
            pl.BlockSpec((c, c), rep), pl.BlockSpec((c,), rep1),
            pl.BlockSpec((8, 8), rep), pl.BlockSpec((8,), rep1),
            pl.BlockSpec((8, c), rep), pl.BlockSpec((c,), rep1),
            pl.BlockSpec((c, s), rep), pl.BlockSpec((s,), rep1),
            pl.BlockSpec((s, s), rep), pl.BlockSpec((s,), rep1),
            pl.BlockSpec((c, c), rep), pl.BlockSpec((c,), rep1),
        ],
        out_specs=pl.BlockSpec((b, c), lambda i: (i, 0)),
        out_shape=jax.ShapeDtypeStruct((n, c), jnp.float32),
    )(x, y, g, p8,
      lp["q"]["w"], lp["q"]["b"], lp["k"]["w"], lp["k"]["b"],
      lp["v"]["w"], lp["v"]["b"], p1w, p1b, p2w, lp["p2"]["b"],
      lp["a1"]["w"], lp["a1"]["b"], lp["a2"]["w"], lp["a2"]["b"],
      prm["lin2"]["w"], prm["lin2"]["b"])


def _tdown_body(ns, cin, cout, b,
                pg_ref, xg_ref, wp_ref, wx_ref, b_ref, o_ref):
    f32 = jnp.float32
    gfull = pg_ref[...]
    P = (gfull[:, :, cin:cin + 8] - xg_ref[...][:, None, :]).reshape(b * ns, 8)
    G = gfull[:, :, :cin].reshape(b * ns, cin)
    g = (jnp.dot(P, wp_ref[...], preferred_element_type=f32)
         + jnp.dot(G, wx_ref[...], preferred_element_type=f32) + b_ref[...])
    g = jnp.maximum(g, 0.0)
    o_ref[...] = jnp.max(g.reshape(b, ns, cout), axis=1)


def t_down(p, x, prm, stride, ns):
    if stride == 1:
        return p, jax.nn.relu(lin(x, prm))
    m = x.shape[0] // stride
    cin = x.shape[1]
    cout = prm["w"].shape[1]
    pn = p[:m]
    idx = knn_idx(pn, p, ns)
    tab = jnp.pad(jnp.concatenate([x, p], axis=1), ((0, 0), (0, 5)))
    g = tab[idx]
    pn8 = jnp.pad(pn, ((0, 0), (0, 5)))
    wp = jnp.pad(prm["w"][:3], ((0, 5), (0, 0)))
    wx = prm["w"][3:]
    b = min(m, 512)
    body = functools.partial(_tdown_body, ns, cin, cout, b)
    rep = lambda i: (0, 0)
    rep1 = lambda i: (0,)
    g = pl.pallas_call(
        body,
        grid=(m // b,),
        in_specs=[
            pl.BlockSpec((b, ns, cin + 8), lambda i: (i, 0, 0)),
            pl.BlockSpec((b, 8), lambda i: (i, 0)),
            pl.BlockSpec((8, cout), rep),
            pl.BlockSpec((cin, cout), rep),
            pl.BlockSpec((cout,), rep1),
        ],
        out_specs=pl.BlockSpec((b, cout), lambda i: (i, 0)),
        out_shape=jax.ShapeDtypeStruct((m, cout), jnp.float32),
    )(g, pn8, wp, wx, prm["b"])
    return pn, g


def _tup_body(cf, b, xf_ref, xg_ref, w_ref, l1w_ref, l1b_ref, o_ref):
    f32 = jnp.float32
    x1 = jnp.dot(xf_ref[...], l1w_ref[...], preferred_element_type=f32) + l1b_ref[...]
    xg = xg_ref[...][:, :, :cf]
    w = w_ref[...]
    o_ref[...] = x1 + jnp.sum(xg * w[:, :, None], axis=1)


def t_up(pf, xf, pc, xc, prm):
    n, cf = xf.shape[0], prm["l1"]["w"].shape[1]
    x2 = lin(xc, prm["l2"])
    idx = knn_idx(pf, pc, 3)
    tab = jnp.pad(jnp.concatenate([x2, pc], axis=1), ((0, 0), (0, 5)))
    g3 = tab[idx]
    pcg = g3[:, :, cf:cf + 3]
    d = jnp.sum((pf[:, None, :] - pcg) ** 2, -1)
    w = 1.0 / (d + 1e-8)
    w = w / jnp.sum(w, -1, keepdims=True)
    w = jnp.pad(w, ((0, 0), (0, 5)))
    xg = jnp.pad(g3, ((0, 0), (0, 5), (0, 0)))
    b = min(n, 512)
    body = functools.partial(_tup_body, cf, b)
    rep = lambda i: (0, 0)
    rep1 = lambda i: (0,)
    return pl.pallas_call(
        body,
        grid=(n // b,),
        in_specs=[
            pl.BlockSpec((b, cf), lambda i: (i, 0)),
            pl.BlockSpec((b, 8, cf + 8), lambda i: (i, 0, 0)),
            pl.BlockSpec((b, 8), lambda i: (i, 0)),
            pl.BlockSpec((cf, cf), rep),
            pl.BlockSpec((cf,), rep1),
        ],
        out_specs=pl.BlockSpec((b, cf), lambda i: (i, 0)),
        out_shape=jax.ShapeDtypeStruct((n, cf), jnp.float32),
    )(xf, xg, w, prm["l1"]["w"], prm["l1"]["b"])


def t_up_head(x, prm):
    x1 = lin(x, prm["l1"])
    g = lin(jnp.mean(x, axis=0, keepdims=True), prm["l2"])
    return x1 + g


def _cls_kernel(x_ref, w1_ref, b1_ref, w2_ref, b2_ref, o_ref):
    y = jnp.maximum(jnp.dot(x_ref[...], w1_ref[...],
                            preferred_element_type=jnp.float32) + b1_ref[...], 0.0)
    o_ref[...] = jnp.dot(y, w2_ref[...],
                         preferred_element_type=jnp.float32) + b2_ref[...]


def cls_head(x, p1, p2):
    n, c = x.shape
    nc = NUM_CLASSES
    blk = 2048
    return pl.pallas_call(
        _cls_kernel,
        grid=(n // blk,),
        in_specs=[
            pl.BlockSpec((blk, c), lambda i: (i, 0)),
            pl.BlockSpec((c, c), lambda i: (0, 0)),
            pl.BlockSpec((c,), lambda i: (0,)),
            pl.BlockSpec((c, nc), lambda i: (0, 0)),
            pl.BlockSpec((nc,), lambda i: (0,)),
        ],
        out_specs=pl.BlockSpec((blk, nc), lambda i: (i, 0)),
        out_shape=jax.ShapeDtypeStruct((n, nc), jnp.float32),
    )(x, p1["w"], p1["b"], p2["w"], p2["b"])


def kernel(coord, feat, offset, params):
    p1, x1 = t_down(coord, feat, params["enc1_td"], 1, NSAMPLE[0])
    idxs = [knn_idx(coord, coord, NSAMPLE[0])] + [None] * 4
    x1 = pt_block(p1, x1, params["enc1_blk"], NSAMPLE[0], idxs[0])
    ps, xs = [p1], [x1]
    pc, xc = p1, x1
    for i in range(1, 5):
        pc, xc = t_down(pc, xc, params["enc%d_td" % (i + 1)], STRIDE[i], NSAMPLE[i])
        idxs[i] = knn_idx(pc, pc, NSAMPLE[i])
        xc = pt_block(pc, xc, params["enc%d_blk" % (i + 1)], NSAMPLE[i], idxs[i])
        ps.append(pc)
        xs.append(xc)
    p1, p2, p3, p4, p5 = ps
    x1, x2, x3, x4, x5 = xs
    x5 = pt_block(p5, t_up_head(x5, params["dec5_tu"]), params["dec5_blk"], NSAMPLE[4], idxs[4])
    x4 = pt_block(p4, t_up(p4, x4, p5, x5, params["dec4_tu"]), params["dec4_blk"], NSAMPLE[3], idxs[3])
    x3 = pt_block(p3, t_up(p3, x3, p4, x4, params["dec3_tu"]), params["dec3_blk"], NSAMPLE[2], idxs[2])
    x2 = pt_block(p2, t_up(p2, x2, p3, x3, params["dec2_tu"]), params["dec2_blk"], NSAMPLE[1], idxs[1])
    x1 = pt_block(p1, t_up(p1, x1, p2, x2, params["dec1_tu"]), params["dec1_blk"], NSAMPLE[0], idxs[0])
    return cls_head(x1, params["cls1"], params["cls2"])


# speedup vs baseline: 1.3335x; 1.0287x over previous
import functools

import jax
import jax.numpy as jnp
import numpy as np
from jax.experimental import pallas as pl

PLANES = [32, 64, 128, 256, 512]
STRIDE = [1, 4, 4, 4, 4]
NSAMPLE = [8, 16, 16, 16, 16]
SHARE = 8
N0 = 16384
IN_CH = 6
NUM_CLASSES = 50


def lin(x, p):
    return x @ p["w"] + p["b"]


def _knn_body(k, n, qb, q_ref, kpt_ref, o_ref):
    q = q_ref[...]
    kpt = kpt_ref[...]
    kn2 = jnp.sum(kpt * kpt, axis=0)[None, :]
    q2 = jnp.sum(q * q, axis=1, keepdims=True)
    d = q2 + kn2 - 2.0 * jnp.dot(q, kpt, preferred_element_type=jnp.float32)
    iota = jax.lax.broadcasted_iota(jnp.int32, (qb, n), 1)
    big = jnp.float32(jnp.inf)
    for j in range(k):
        ij = jnp.argmin(d, axis=1).astype(jnp.int32)[:, None]
        o_ref[:, j] = ij[:, 0]
        d = jnp.where(iota == ij, big, d)


def knn_idx(q, kp, k):
    M, N = q.shape[0], kp.shape[0]
    QB = min(M, 256)
    qpad = jnp.pad(q, ((0, 0), (0, 5)))
    kpt = jnp.pad(kp, ((0, 0), (0, 5))).T
    body = functools.partial(_knn_body, k, N, QB)
    return pl.pallas_call(
        body,
        grid=(M // QB,),
        in_specs=[
            pl.BlockSpec((QB, 8), lambda i: (i, 0)),
            pl.BlockSpec((8, N), lambda i: (0, 0)),
        ],
        out_specs=pl.BlockSpec((QB, k), lambda i: (i, 0)),
        out_shape=jax.ShapeDtypeStruct((M, k), jnp.int32),
    )(qpad, kpt)


def _ptb_body(ns, c, s, b,
              x_ref, y_ref, yg_ref, pr_ref,
              qw_ref, qb_ref, kw_ref, kb_ref, vw_ref, vb_ref,
              p1w_ref, p1b_ref, p2w_ref, p2b_ref,
              a1w_ref, a1b_ref, a2w_ref, a2b_ref,
              l2w_ref, l2b_ref, o_ref):
    f32 = jnp.float32
    q = jnp.dot(y_ref[...], qw_ref[...], preferred_element_type=f32) + qb_ref[...]
    gfull = yg_ref[...]
    G = gfull[:, :, :c].reshape(b * ns, c)
    P = (gfull[:, :, c:c + 8] - pr_ref[...][:, None, :]).reshape(b * ns, 8)
    kf = jnp.dot(G, kw_ref[...], preferred_element_type=f32) + kb_ref[...]
    v = jnp.dot(G, vw_ref[...], preferred_element_type=f32) + vb_ref[...]
    pe = jnp.maximum(
        jnp.dot(P, p1w_ref[...], preferred_element_type=f32) + p1b_ref[...], 0.0)
    pe = jnp.dot(pe, p2w_ref[...], preferred_element_type=f32) + p2b_ref[...]
    w3 = q[:, None, :] - kf.reshape(b, ns, c) + pe.reshape(b, ns, c)
    h = jnp.maximum(
        jnp.dot(w3.reshape(b * ns, c), a1w_ref[...], preferred_element_type=f32)
        + a1b_ref[...], 0.0)
    h = jnp.dot(h, a2w_ref[...], preferred_element_type=f32) + a2b_ref[...]
    h3 = h.reshape(b, ns, s)
    m = jnp.max(h3, axis=1, keepdims=True)
    e = jnp.exp(h3 - m)
    z = jnp.sum(e, axis=1, keepdims=True)
    a = e / z
    af = jnp.concatenate([a] * SHARE, axis=2)
    vpe = v.reshape(b, ns, c) + pe.reshape(b, ns, c)
    attn = jnp.sum(vpe * af, axis=1)
    z2 = jnp.maximum(attn, 0.0)
    out = jnp.dot(z2, l2w_ref[...], preferred_element_type=f32) + l2b_ref[...]
    o_ref[...] = jnp.maximum(x_ref[...] + out, 0.0)


def pt_block(p, x, prm, ns, idx):
    n, c = x.shape
    s = c // SHARE
    lp = prm["layer"]
    y = jax.nn.relu(lin(x, prm["lin1"]))
    tab = jnp.pad(jnp.concatenate([y, p], axis=1), ((0, 0), (0, 5)))
    g = tab[idx]
    p8 = jnp.pad(p, ((0, 0), (0, 5)))
    p1w = jnp.pad(lp["p1"]["w"], ((0, 5), (0, 5)))
    p1b = jnp.pad(lp["p1"]["b"], ((0, 5),))
    p2w = jnp.pad(lp["p2"]["w"], ((0, 5), (0, 0)))
    b = min(n, 512)
    body = functools.partial(_ptb_body, ns, c, s, b)
    rep = lambda i: (0, 0)
    rep1 = lambda i: (0,)
    return pl.pallas_call(
        body,
        grid=(n // b,),
        in_specs=[
            pl.BlockSpec((b, c), lambda i: (i, 0)),
            pl.BlockSpec((b, c), lambda i: (i, 0)),
            pl.BlockSpec((b, ns, c + 8), lambda i: (i, 0, 0)),
            pl.BlockSpec((b, 8), lambda i: (i, 0)),
            pl.BlockSpec((c, c), rep), pl.BlockSpec((c,), rep1),
            pl.BlockSpec((c, c), rep), pl.BlockSpec((c,), rep1),
            pl.BlockSpec((c, c), rep), pl.BlockSpec((c,), rep1),
            pl.BlockSpec((8, 8), rep), pl.BlockSpec((8,), rep1),
            pl.BlockSpec((8, c), rep), pl.BlockSpec((c,), rep1),
            pl.BlockSpec((c, s), rep), pl.BlockSpec((s,), rep1),
            pl.BlockSpec((s, s), rep), pl.BlockSpec((s,), rep1),
            pl.BlockSpec((c, c), rep), pl.BlockSpec((c,), rep1),
        ],
        out_specs=pl.BlockSpec((b, c), lambda i: (i, 0)),
        out_shape=jax.ShapeDtypeStruct((n, c), jnp.float32),
    )(x, y, g, p8,
      lp["q"]["w"], lp["q"]["b"], lp["k"]["w"], lp["k"]["b"],
      lp["v"]["w"], lp["v"]["b"], p1w, p1b, p2w, lp["p2"]["b"],
      lp["a1"]["w"], lp["a1"]["b"], lp["a2"]["w"], lp["a2"]["b"],
      prm["lin2"]["w"], prm["lin2"]["b"])


def _tdown_body(ns, cin, cout, b,
                pg_ref, xg_ref, wp_ref, wx_ref, b_ref, o_ref):
    f32 = jnp.float32
    gfull = pg_ref[...]
    P = (gfull[:, :, cin:cin + 8] - xg_ref[...][:, None, :]).reshape(b * ns, 8)
    G = gfull[:, :, :cin].reshape(b * ns, cin)
    g = (jnp.dot(P, wp_ref[...], preferred_element_type=f32)
         + jnp.dot(G, wx_ref[...], preferred_element_type=f32) + b_ref[...])
    g = jnp.maximum(g, 0.0)
    o_ref[...] = jnp.max(g.reshape(b, ns, cout), axis=1)


def t_down(p, x, prm, stride, ns):
    if stride == 1:
        return p, jax.nn.relu(lin(x, prm))
    m = x.shape[0] // stride
    cin = x.shape[1]
    cout = prm["w"].shape[1]
    pn = p[:m]
    idx = knn_idx(pn, p, ns)
    tab = jnp.pad(jnp.concatenate([x, p], axis=1), ((0, 0), (0, 5)))
    g = tab[idx]
    pn8 = jnp.pad(pn, ((0, 0), (0, 5)))
    wp = jnp.pad(prm["w"][:3], ((0, 5), (0, 0)))
    wx = prm["w"][3:]
    b = min(m, 512)
    body = functools.partial(_tdown_body, ns, cin, cout, b)
    rep = lambda i: (0, 0)
    rep1 = lambda i: (0,)
    g = pl.pallas_call(
        body,
        grid=(m // b,),
        in_specs=[
            pl.BlockSpec((b, ns, cin + 8), lambda i: (i, 0, 0)),
            pl.BlockSpec((b, 8), lambda i: (i, 0)),
            pl.BlockSpec((8, cout), rep),
            pl.BlockSpec((cin, cout), rep),
            pl.BlockSpec((cout,), rep1),
        ],
        out_specs=pl.BlockSpec((b, cout), lambda i: (i, 0)),
        out_shape=jax.ShapeDtypeStruct((m, cout), jnp.float32),
    )(g, pn8, wp, wx, prm["b"])
    return pn, g


def _tup_body(cf, b, xf_ref, xg_ref, w_ref, l1w_ref, l1b_ref, o_ref):
    f32 = jnp.float32
    x1 = jnp.dot(xf_ref[...], l1w_ref[...], preferred_element_type=f32) + l1b_ref[...]
    xg = xg_ref[...][:, :, :cf]
    w = w_ref[...]
    o_ref[...] = x1 + jnp.sum(xg * w[:, :, None], axis=1)


def t_up(pf, xf, pc, xc, prm):
    n, cf = xf.shape[0], prm["l1"]["w"].shape[1]
    x2 = lin(xc, prm["l2"])
    idx = knn_idx(pf, pc, 3)
    tab = jnp.pad(jnp.concatenate([x2, pc], axis=1), ((0, 0), (0, 5)))
    g3 = tab[idx]
    pcg = g3[:, :, cf:cf + 3]
    d = jnp.sum((pf[:, None, :] - pcg) ** 2, -1)
    w = 1.0 / (d + 1e-8)
    w = w / jnp.sum(w, -1, keepdims=True)
    w = jnp.pad(w, ((0, 0), (0, 5)))
    xg = jnp.pad(g3, ((0, 0), (0, 5), (0, 0)))
    b = min(n, 512)
    body = functools.partial(_tup_body, cf, b)
    rep = lambda i: (0, 0)
    rep1 = lambda i: (0,)
    return pl.pallas_call(
        body,
        grid=(n // b,),
        in_specs=[
            pl.BlockSpec((b, cf), lambda i: (i, 0)),
            pl.BlockSpec((b, 8, cf + 8), lambda i: (i, 0, 0)),
            pl.BlockSpec((b, 8), lambda i: (i, 0)),
            pl.BlockSpec((cf, cf), rep),
            pl.BlockSpec((cf,), rep1),
        ],
        out_specs=pl.BlockSpec((b, cf), lambda i: (i, 0)),
        out_shape=jax.ShapeDtypeStruct((n, cf), jnp.float32),
    )(xf, xg, w, prm["l1"]["w"], prm["l1"]["b"])


def t_up_head(x, prm):
    x1 = lin(x, prm["l1"])
    g = lin(jnp.mean(x, axis=0, keepdims=True), prm["l2"])
    return x1 + g


def _cls_kernel(x_ref, w1_ref, b1_ref, w2_ref, b2_ref, o_ref):
    y = jnp.maximum(jnp.dot(x_ref[...], w1_ref[...],
                            preferred_element_type=jnp.float32) + b1_ref[...], 0.0)
    o_ref[...] = jnp.dot(y, w2_ref[...],
                         preferred_element_type=jnp.float32) + b2_ref[...]


def cls_head(x, p1, p2):
    n, c = x.shape
    nc = NUM_CLASSES
    blk = 2048
    return pl.pallas_call(
        _cls_kernel,
        grid=(n // blk,),
        in_specs=[
            pl.BlockSpec((blk, c), lambda i: (i, 0)),
            pl.BlockSpec((c, c), lambda i: (0, 0)),
            pl.BlockSpec((c,), lambda i: (0,)),
            pl.BlockSpec((c, nc), lambda i: (0, 0)),
            pl.BlockSpec((nc,), lambda i: (0,)),
        ],
        out_specs=pl.BlockSpec((blk, nc), lambda i: (i, 0)),
        out_shape=jax.ShapeDtypeStruct((n, nc), jnp.float32),
    )(x, p1["w"], p1["b"], p2["w"], p2["b"])


def kernel(coord, feat, offset, params):
    p1, x1 = t_down(coord, feat, params["enc1_td"], 1, NSAMPLE[0])
    idxs = [knn_idx(coord, coord, NSAMPLE[0])] + [None] * 4
    x1 = pt_block(p1, x1, params["enc1_blk"], NSAMPLE[0], idxs[0])
    ps, xs = [p1], [x1]
    pc, xc = p1, x1
    for i in range(1, 5):
        pc, xc = t_down(pc, xc, params["enc%d_td" % (i + 1)], STRIDE[i], NSAMPLE[i])
        idxs[i] = knn_idx(pc, pc, NSAMPLE[i])
        xc = pt_block(pc, xc, params["enc%d_blk" % (i + 1)], NSAMPLE[i], idxs[i])
        ps.append(pc)
        xs.append(xc)
    p1, p2, p3, p4, p5 = ps
    x1, x2, x3, x4, x5 = xs
    x5 = pt_block(p5, t_up_head(x5, params["dec5_tu"]), params["dec5_blk"], NSAMPLE[4], idxs[4])
    x4 = pt_block(p4, t_up(p4, x4, p5, x5, params["dec4_tu"]), params["dec4_blk"], NSAMPLE[3], idxs[3])
    x3 = pt_block(p3, t_up(p3, x3, p4, x4, params["dec3_tu"]), params["dec3_blk"], NSAMPLE[2], idxs[2])
    x2 = pt_block(p2, t_up(p2, x2, p3, x3, params["dec2_tu"]), params["dec2_blk"], NSAMPLE[1], idxs[1])
    x1 = pt_block(p1, t_up(p1, x1, p2, x2, params["dec1_tu"]), params["dec1_blk"], NSAMPLE[0], idxs[0])
    return cls_head(x1, params["cls1"], params["cls2"])
